# Initial kernel scaffold; baseline (speedup 1.0000x reference)
import jax
import jax.numpy as jnp
from jax.experimental import pallas as pl


def kernel(prediction, label):
    raise NotImplementedError("write your pallas kernel here")


# speedup vs baseline: 10.7230x; 10.7230x over previous
import functools

import jax
import jax.numpy as jnp
from jax import lax
from jax.experimental import pallas as pl
from jax.experimental.pallas import tpu as pltpu
from jax.experimental.pallas import tpu_sc as plsc

D_VAR_ = 0.5
NLAB_PAD = 128
LANES = 16
NTILES = 32
BM = 4096


def _hinge_body(x_ref, o_ref):
    x = x_ref[...]
    center = jnp.mean(x, axis=1, keepdims=True)
    d = x - center
    s = jnp.sum(d * d, axis=1)
    dist = jnp.sqrt(s)
    h = jnp.maximum(dist - D_VAR_, 0.0)
    o_ref[...] = h * h


def _hinge(prediction):
    B, C, M = prediction.shape
    return pl.pallas_call(
        _hinge_body,
        grid=(M // BM,),
        in_specs=[pl.BlockSpec((B, C, BM), lambda m: (0, 0, m))],
        out_specs=pl.BlockSpec((B, BM), lambda m: (0, m)),
        out_shape=jax.ShapeDtypeStruct((B, M), jnp.float32),
    )(prediction)


def _hist_body(hinge_hbm, label_hbm, out_s, out_c, hin_v, lab_v, acc_s, acc_c):
    c = lax.axis_index("c")
    s = lax.axis_index("s")
    wid = s * 2 + c
    chunk = hin_v.shape[0]
    base = wid * chunk

    pltpu.sync_copy(hinge_hbm.at[pl.ds(base, chunk)], hin_v)
    pltpu.sync_copy(label_hbm.at[pl.ds(base, chunk)], lab_v)

    zf = jnp.zeros((LANES,), jnp.float32)
    for j in range(LANES * NLAB_PAD // LANES):
        acc_s[pl.ds(j * LANES, LANES)] = zf
        acc_c[pl.ds(j * LANES, LANES)] = zf

    lane_off = lax.iota(jnp.int32, LANES) * NLAB_PAD
    ones = jnp.ones((LANES,), jnp.float32)

    def body(i, carry):
        lab = lab_v[pl.ds(i * LANES, LANES)]
        hin = hin_v[pl.ds(i * LANES, LANES)]
        idx = lane_off + lab
        plsc.addupdate_scatter(acc_s, [idx], hin)
        plsc.addupdate_scatter(acc_c, [idx], ones)
        return carry

    lax.fori_loop(0, chunk // LANES, body, 0)

    pltpu.sync_copy(acc_s, out_s.at[wid])
    pltpu.sync_copy(acc_c, out_c.at[wid])


def _hist(hinge_flat, label_flat):
    n = hinge_flat.shape[0]
    chunk = n // NTILES
    mesh = plsc.VectorSubcoreMesh(core_axis_name="c", subcore_axis_name="s")
    f32 = jnp.float32
    out_t = (jax.ShapeDtypeStruct((NTILES, LANES * NLAB_PAD), f32),
             jax.ShapeDtypeStruct((NTILES, LANES * NLAB_PAD), f32))
    run = pl.kernel(
        _hist_body,
        out_type=out_t,
        mesh=mesh,
        compiler_params=pltpu.CompilerParams(needs_layout_passes=False),
        scratch_types=[
            pltpu.VMEM((chunk,), f32),
            pltpu.VMEM((chunk,), jnp.int32),
            pltpu.VMEM((LANES * NLAB_PAD,), f32),
            pltpu.VMEM((LANES * NLAB_PAD,), f32),
        ],
    )
    return run(hinge_flat, label_flat)


def _combine_body(s_ref, c_ref, o_ref):
    s = jnp.sum(s_ref[...], axis=1)
    c = jnp.sum(c_ref[...], axis=1)
    present = c > 0.0
    denom = jnp.where(present, c, 1.0)
    terms = jnp.where(present, s / denom, 0.0)
    inst = jnp.sum(terms, axis=1)
    nu = jnp.sum(present.astype(jnp.float32), axis=1)
    o_ref[...] = jnp.reshape(jnp.sum(inst / nu), (1, 1))


def _combine(sums, counts):
    return pl.pallas_call(
        _combine_body,
        out_shape=jax.ShapeDtypeStruct((1, 1), jnp.float32),
    )(sums, counts)


@jax.jit
def kernel(prediction, label):
    B, C, M = prediction.shape
    h = _hinge(prediction)
    sums, counts = _hist(h.reshape(-1), label.reshape(-1))
    tiles_per_b = NTILES // B
    sums = sums.reshape(B, tiles_per_b * LANES, NLAB_PAD)
    counts = counts.reshape(B, tiles_per_b * LANES, NLAB_PAD)
    out = _combine(sums, counts)
    return out[0, 0]


# speedup vs baseline: 10.7254x; 1.0002x over previous
import functools

import jax
import jax.numpy as jnp
from jax import lax
from jax.experimental import pallas as pl
from jax.experimental.pallas import tpu as pltpu
from jax.experimental.pallas import tpu_sc as plsc

D_VAR_ = 0.5
NLAB_PAD = 128
LANES = 16
NTILES = 32
BM = 4096


def _hinge_body(x_ref, o_ref):
    x = x_ref[...]
    c = x.shape[1]
    s1 = jnp.sum(x, axis=1)
    s2 = jnp.sum(x * x, axis=1)
    var = jnp.maximum(s2 - s1 * s1 * (1.0 / c), 0.0)
    dist = jnp.sqrt(var)
    h = jnp.maximum(dist - D_VAR_, 0.0)
    o_ref[...] = h * h


def _hinge(prediction):
    B, C, M = prediction.shape
    return pl.pallas_call(
        _hinge_body,
        grid=(M // BM,),
        in_specs=[pl.BlockSpec((B, C, BM), lambda m: (0, 0, m))],
        out_specs=pl.BlockSpec((B, BM), lambda m: (0, m)),
        out_shape=jax.ShapeDtypeStruct((B, M), jnp.float32),
    )(prediction)


def _hist_body(hinge_hbm, label_hbm, out_s, out_c, hin_v, lab_v, acc_s, acc_c):
    c = lax.axis_index("c")
    s = lax.axis_index("s")
    wid = s * 2 + c
    chunk = hin_v.shape[0]
    base = wid * chunk

    pltpu.sync_copy(hinge_hbm.at[pl.ds(base, chunk)], hin_v)
    pltpu.sync_copy(label_hbm.at[pl.ds(base, chunk)], lab_v)

    zf = jnp.zeros((LANES,), jnp.float32)
    for j in range(LANES * NLAB_PAD // LANES):
        acc_s[pl.ds(j * LANES, LANES)] = zf
        acc_c[pl.ds(j * LANES, LANES)] = zf

    lane_off = lax.iota(jnp.int32, LANES) * NLAB_PAD
    ones = jnp.ones((LANES,), jnp.float32)

    UNROLL = 8

    def body(i, carry):
        for u in range(UNROLL):
            off = (i * UNROLL + u) * LANES
            lab = lab_v[pl.ds(off, LANES)]
            hin = hin_v[pl.ds(off, LANES)]
            idx = lane_off + lab
            plsc.addupdate_scatter(acc_s, [idx], hin)
            plsc.addupdate_scatter(acc_c, [idx], ones)
        return carry

    lax.fori_loop(0, chunk // (LANES * UNROLL), body, 0)

    pltpu.sync_copy(acc_s, out_s.at[wid])
    pltpu.sync_copy(acc_c, out_c.at[wid])


def _hist(hinge_flat, label_flat):
    n = hinge_flat.shape[0]
    chunk = n // NTILES
    mesh = plsc.VectorSubcoreMesh(core_axis_name="c", subcore_axis_name="s")
    f32 = jnp.float32
    out_t = (jax.ShapeDtypeStruct((NTILES, LANES * NLAB_PAD), f32),
             jax.ShapeDtypeStruct((NTILES, LANES * NLAB_PAD), f32))
    run = pl.kernel(
        _hist_body,
        out_type=out_t,
        mesh=mesh,
        compiler_params=pltpu.CompilerParams(needs_layout_passes=False),
        scratch_types=[
            pltpu.VMEM((chunk,), f32),
            pltpu.VMEM((chunk,), jnp.int32),
            pltpu.VMEM((LANES * NLAB_PAD,), f32),
            pltpu.VMEM((LANES * NLAB_PAD,), f32),
        ],
    )
    return run(hinge_flat, label_flat)


def _combine_body(s_ref, c_ref, o_ref):
    s = jnp.sum(s_ref[...], axis=1)
    c = jnp.sum(c_ref[...], axis=1)
    present = c > 0.0
    denom = jnp.where(present, c, 1.0)
    terms = jnp.where(present, s / denom, 0.0)
    inst = jnp.sum(terms, axis=1)
    nu = jnp.sum(present.astype(jnp.float32), axis=1)
    o_ref[...] = jnp.reshape(jnp.sum(inst / nu), (1, 1))


def _combine(sums, counts):
    return pl.pallas_call(
        _combine_body,
        out_shape=jax.ShapeDtypeStruct((1, 1), jnp.float32),
    )(sums, counts)


@jax.jit
def kernel(prediction, label):
    B, C, M = prediction.shape
    h = _hinge(prediction)
    sums, counts = _hist(h.reshape(-1), label.reshape(-1))
    tiles_per_b = NTILES // B
    sums = sums.reshape(B, tiles_per_b * LANES, NLAB_PAD)
    counts = counts.reshape(B, tiles_per_b * LANES, NLAB_PAD)
    out = _combine(sums, counts)
    return out[0, 0]
